# baseline (speedup 1.0000x reference)
import functools

import jax
import jax.numpy as jnp
from jax import lax
from jax.experimental import pallas as pl
from jax.experimental.pallas import tpu as pltpu
from jax.experimental.pallas import tpu_sc as plsc

ENTITY_VOCAB = 1000000
EMBED_DIM = 64
BATCH = 4096
HIST = 50

_INFO = plsc.get_sparse_core_info()
_NC = _INFO.num_cores
_NS = _INFO.num_subcores
_NW = _NC * _NS
_B = BATCH * HIST
_BPW = _B // _NW
_CHUNK = 128
_NCHUNK = _BPW // _CHUNK


def _gather_body(idx_hbm, table_hbm, out_hbm, idx_v, rows_v, sem):
    wid = lax.axis_index("s") * _NC + lax.axis_index("c")
    base = wid * _BPW
    pltpu.sync_copy(idx_hbm.at[wid], idx_v)

    def step(j, carry):
        pltpu.async_copy(table_hbm.at[idx_v.at[j]], rows_v, sem).wait()
        pltpu.sync_copy(rows_v, out_hbm.at[pl.ds(base + j * _CHUNK, _CHUNK)])
        return carry

    lax.fori_loop(0, _NCHUNK, step, 0)


@jax.jit
def _sc_gather(idx, table):
    mesh = plsc.VectorSubcoreMesh(core_axis_name="c", subcore_axis_name="s")
    fn = pl.kernel(
        _gather_body,
        mesh=mesh,
        out_type=jax.ShapeDtypeStruct((_B, EMBED_DIM), jnp.float32),
        scratch_types=[
            pltpu.VMEM((_NCHUNK, _CHUNK), jnp.int32),
            pltpu.VMEM((_CHUNK, EMBED_DIM), jnp.float32),
            pltpu.SemaphoreType.DMA,
        ],
        compiler_params=pltpu.CompilerParams(use_tc_tiling_on_sc=False),
    )
    return fn(idx, table)


def kernel(entities, entity_table):
    idx = entities.reshape(_NW, _NCHUNK, _CHUNK)
    out = _sc_gather(idx, entity_table)
    return out.reshape(BATCH, HIST, EMBED_DIM)


# speedup vs baseline: 1.0449x; 1.0449x over previous
import functools

import jax
import jax.numpy as jnp
from jax import lax
from jax.experimental import pallas as pl
from jax.experimental.pallas import tpu as pltpu
from jax.experimental.pallas import tpu_sc as plsc

ENTITY_VOCAB = 1000000
EMBED_DIM = 64
BATCH = 4096
HIST = 50

_INFO = plsc.get_sparse_core_info()
_NC = _INFO.num_cores
_NS = _INFO.num_subcores
_NW = _NC * _NS
_B = BATCH * HIST
_BPW = _B // _NW
_CHUNK = 128
_NCHUNK = _BPW // _CHUNK
_NBUF = 10


def _gather_body(idx_hbm, table_hbm, out_hbm, idx_v, rows_v, sem_in, sem_out):
    wid = lax.axis_index("s") * _NC + lax.axis_index("c")
    base = wid * _BPW
    pltpu.sync_copy(idx_hbm.at[wid], idx_v)

    def gather(j, b):
        return pltpu.make_async_copy(
            table_hbm.at[idx_v.at[j]], rows_v.at[b], sem_in.at[b])

    def put(j, b):
        return pltpu.make_async_copy(
            rows_v.at[b], out_hbm.at[pl.ds(base + j * _CHUNK, _CHUNK)],
            sem_out.at[b])

    for b in range(_NBUF):
        gather(b, b).start()

    def outer(g, carry):
        for b in range(_NBUF):
            j = g * _NBUF + b
            gather(j, b).wait()
            put(j, b).start()
            put(j, b).wait()
            gather(j + _NBUF, b).start()
        return carry

    lax.fori_loop(0, _NCHUNK // _NBUF - 1, outer, 0)

    for b in range(_NBUF):
        j = _NCHUNK - _NBUF + b
        gather(j, b).wait()
        put(j, b).start()
    for b in range(_NBUF):
        put(_NCHUNK - _NBUF + b, b).wait()


@jax.jit
def _sc_gather(idx, table):
    mesh = plsc.VectorSubcoreMesh(core_axis_name="c", subcore_axis_name="s")
    fn = pl.kernel(
        _gather_body,
        mesh=mesh,
        out_type=jax.ShapeDtypeStruct((_B, EMBED_DIM), jnp.float32),
        scratch_types=[
            pltpu.VMEM((_NCHUNK, _CHUNK), jnp.int32),
            pltpu.VMEM((_NBUF, _CHUNK, EMBED_DIM), jnp.float32),
            pltpu.SemaphoreType.DMA((_NBUF,)),
            pltpu.SemaphoreType.DMA((_NBUF,)),
        ],
        compiler_params=pltpu.CompilerParams(use_tc_tiling_on_sc=False),
    )
    return fn(idx, table)


def kernel(entities, entity_table):
    idx = entities.reshape(_NW, _NCHUNK, _CHUNK)
    out = _sc_gather(idx, entity_table)
    return out.reshape(BATCH, HIST, EMBED_DIM)
